# baseline (speedup 1.0000x reference)
import functools

import jax
import jax.numpy as jnp
from jax import lax
from jax.experimental import pallas as pl
from jax.experimental.pallas import tpu as pltpu
from jax.experimental.pallas import tpu_sc as plsc

B = 4096
L = 200
EMB = 64
VLM = 768
TXT = 512
VOCAB = 1000000

NC = 2
NS = 16
NW = NC * NS
BPW = B // NW
NBUF = 4
C0 = 128
C1 = L - C0

TBLK = 8192


NBLK = (VOCAB + 2 * TBLK - 1) // (2 * TBLK)
TROWS = 2 * NBLK * TBLK


def _detile_table(embT):
    def body(ina_ref, inb_ref, o_ref):
        o_ref[...] = jnp.concatenate(
            [ina_ref[...].T, inb_ref[...].T], axis=1
        )

    return pl.pallas_call(
        body,
        grid=(NBLK,),
        in_specs=[
            pl.BlockSpec((EMB, TBLK), lambda i: (0, 2 * i)),
            pl.BlockSpec((EMB, TBLK), lambda i: (0, jnp.minimum(2 * i + 1, 122))),
        ],
        out_specs=pl.BlockSpec((TBLK, 2 * EMB), lambda i: (i, 0)),
        out_shape=jax.ShapeDtypeStruct((NBLK * TBLK, 2 * EMB), jnp.float32),
    )(embT, embT)


def _transpose_out(o2):
    RB, CB = 512, 2560
    GI, GJ = B // RB, (L * EMB) // CB
    N = GI * GJ

    def body(hbm_ref, o_ref, buf, sem):
        i = pl.program_id(0)
        j = pl.program_id(1)
        flat = i * GJ + j
        p = lax.rem(flat, 2)

        def start_copy(f, slot):
            i2 = f // GJ
            j2 = lax.rem(f, GJ)
            pltpu.make_async_copy(
                hbm_ref.at[pl.ds(i2 * RB, RB), pl.ds(j2 * CB, CB)],
                buf.at[slot],
                sem.at[slot],
            ).start()

        def wait_copy(f, slot):
            i2 = f // GJ
            j2 = lax.rem(f, GJ)
            pltpu.make_async_copy(
                hbm_ref.at[pl.ds(i2 * RB, RB), pl.ds(j2 * CB, CB)],
                buf.at[slot],
                sem.at[slot],
            ).wait()

        @pl.when(flat == 0)
        def _first():
            start_copy(0, 0)

        @pl.when(flat + 1 < N)
        def _pf():
            start_copy(flat + 1, lax.rem(flat + 1, 2))

        wait_copy(flat, p)
        o_ref[...] = buf[p].T

    return pl.pallas_call(
        body,
        grid=(GI, GJ),
        in_specs=[pl.BlockSpec(memory_space=pl.ANY)],
        out_specs=pl.BlockSpec((CB, RB), lambda i, j: (j, i)),
        out_shape=jax.ShapeDtypeStruct((L * EMB, B), jnp.float32),
        scratch_shapes=[
            pltpu.VMEM((2, RB, CB), jnp.float32),
            pltpu.SemaphoreType.DMA((2,)),
        ],
    )(o2)


def _projection(vlm_emb, text_emb, W1a, W1b, b1_2d):
    blk = 512
    grid = (B // blk,)

    def body(vlm_ref, txt_ref, wa_ref, wb_ref, b1_ref, o_ref):
        acc = jnp.dot(vlm_ref[...], wa_ref[...], preferred_element_type=jnp.float32)
        acc = acc + jnp.dot(txt_ref[...], wb_ref[...], preferred_element_type=jnp.float32)
        o_ref[...] = acc + b1_ref[...]

    return pl.pallas_call(
        body,
        grid=grid,
        in_specs=[
            pl.BlockSpec((blk, VLM), lambda i: (i, 0)),
            pl.BlockSpec((blk, TXT), lambda i: (i, 0)),
            pl.BlockSpec((VLM, EMB), lambda i: (0, 0)),
            pl.BlockSpec((TXT, EMB), lambda i: (0, 0)),
            pl.BlockSpec((1, EMB), lambda i: (0, 0)),
        ],
        out_specs=pl.BlockSpec((blk, EMB), lambda i: (i, 0)),
        out_shape=jax.ShapeDtypeStruct((B, EMB), jnp.float32),
    )(vlm_emb, text_emb, W1a, W1b, b1_2d)


def _gather_add(ids, proj, table):
    mesh = plsc.VectorSubcoreMesh(core_axis_name="c", subcore_axis_name="s")

    @functools.partial(
        pl.kernel,
        out_type=jax.ShapeDtypeStruct((B, L, EMB), jnp.float32),
        mesh=mesh,
        scratch_types=[
            pltpu.VMEM((BPW, L), jnp.int32),
            pltpu.VMEM((BPW, EMB), jnp.float32),
            pltpu.VMEM((NBUF, L, EMB), jnp.float32),
            pltpu.SemaphoreType.DMA((NBUF,)),
            pltpu.SemaphoreType.DMA((NBUF,)),
        ],
        compiler_params=pltpu.CompilerParams(use_tc_tiling_on_sc=False),
    )
    def k(ids_hbm, proj_hbm, table_hbm, out_hbm, idx_v, projs_v, rows_v, gsem, osem):
        wid = lax.axis_index("s") * NC + lax.axis_index("c")
        base = wid * BPW

        pltpu.sync_copy(ids_hbm.at[pl.ds(base, BPW)], idx_v)
        pltpu.sync_copy(proj_hbm.at[pl.ds(base, BPW)], projs_v)

        def start_gather(i, buf):
            pltpu.async_copy(
                table_hbm.at[idx_v.at[i, pl.ds(0, C0)]],
                rows_v.at[buf, pl.ds(0, C0)],
                gsem.at[buf],
            )
            pltpu.async_copy(
                table_hbm.at[idx_v.at[i, pl.ds(C0, C1)]],
                rows_v.at[buf, pl.ds(C0, C1)],
                gsem.at[buf],
            )

        def wait_gather(i, buf):
            pltpu.make_async_copy(
                table_hbm.at[idx_v.at[i, pl.ds(0, C0)]],
                rows_v.at[buf, pl.ds(0, C0)],
                gsem.at[buf],
            ).wait()
            pltpu.make_async_copy(
                table_hbm.at[idx_v.at[i, pl.ds(C0, C1)]],
                rows_v.at[buf, pl.ds(C0, C1)],
                gsem.at[buf],
            ).wait()

        def wait_write(i, buf):
            pltpu.make_async_copy(
                rows_v.at[buf], out_hbm.at[base + i], osem.at[buf]
            ).wait()

        start_gather(0, 0)
        start_gather(1, 1)

        @pl.loop(0, BPW // NBUF)
        def _t(t):
            for kk in range(NBUF):
                i = t * NBUF + kk
                buf = kk
                nbuf = (kk + 2) % NBUF

                @pl.when(i + 2 < BPW)
                def _pf():
                    @pl.when(i >= 2)
                    def _drain():
                        wait_write(i - 2, nbuf)

                    start_gather(i + 2, nbuf)

                wait_gather(i, buf)

                pj0 = projs_v[i, pl.ds(0, 16)]
                pj1 = projs_v[i, pl.ds(16, 16)]
                pj2 = projs_v[i, pl.ds(32, 16)]
                pj3 = projs_v[i, pl.ds(48, 16)]

                @pl.loop(0, L // 4)
                def _r(r4):
                    for rr in range(4):
                        r = r4 * 4 + rr
                        rows_v[buf, r, pl.ds(0, 16)] += pj0
                        rows_v[buf, r, pl.ds(16, 16)] += pj1
                        rows_v[buf, r, pl.ds(32, 16)] += pj2
                        rows_v[buf, r, pl.ds(48, 16)] += pj3

                pltpu.async_copy(rows_v.at[buf], out_hbm.at[base + i], osem.at[buf])

        for kk in range(NBUF):
            wait_write(BPW - NBUF + kk, kk)

    return k(ids, proj, table)


def kernel(vlm_emb, text_emb, input_ids, embedding, W1, b1):
    W1a = W1[:VLM]
    W1b = W1[VLM:]
    proj = _projection(vlm_emb, text_emb, W1a, W1b, b1.reshape(1, EMB))

    table_lin = _detile_table(embedding.T).reshape(TROWS, EMB)
    ids32 = input_ids.astype(jnp.int32)
    g = ids32 >> 13
    idx2 = ((g >> 1) << 14) | ((ids32 & 8191) << 1) | (g & 1)
    out = _gather_add(idx2, proj, table_lin)

    flat = jax.lax.optimization_barrier(out.reshape(B * L * EMB))
    o2 = flat.reshape(B, L * EMB)
    o3 = _transpose_out(o2)
    return o3.reshape(L, EMB, B).transpose(2, 0, 1)
